# Initial kernel scaffold; baseline (speedup 1.0000x reference)
import jax
import jax.numpy as jnp
from jax.experimental import pallas as pl


def kernel(x, edge_index, W_l, W_r, b):
    raise NotImplementedError("write your pallas kernel here")


# speedup vs baseline: 7.3535x; 7.3535x over previous
import functools

import jax
import jax.numpy as jnp
from jax import lax
from jax.experimental import pallas as pl
from jax.experimental.pallas import tpu as pltpu
from jax.experimental.pallas import tpu_sc as plsc

N_NODES = 10000
N_EDGES = 320000
D_IN = 128
D_AUG = 144
D_OUT = 128

NC = 2
NS = 16
NW = NC * NS
CHUNK = 80
EDGES_PER_W = N_EDGES // NW
CHUNKS_PER_W = EDGES_PER_W // CHUNK
IDX_BLK = 25
N_IDX_LOADS = CHUNKS_PER_W // IDX_BLK
ROWS_PER_TILE = N_NODES // NS


def _sc_aggregate(x_aug, src2d, dst2d):
    mesh = plsc.VectorSubcoreMesh(
        core_axis_name="c", subcore_axis_name="s", num_cores=NC, num_subcores=NS
    )

    @functools.partial(
        pl.kernel,
        out_type=jax.ShapeDtypeStruct((NC, N_NODES, D_AUG), jnp.float32),
        mesh=mesh,
        compiler_params=pltpu.CompilerParams(use_tc_tiling_on_sc=False),
        scratch_types=[
            pltpu.VMEM_SHARED((N_NODES, D_AUG), jnp.float32),
            pltpu.VMEM((IDX_BLK, CHUNK), jnp.int32),
            pltpu.VMEM((IDX_BLK, CHUNK), jnp.int32),
            pltpu.VMEM((CHUNK, D_AUG), jnp.float32),
            pltpu.SemaphoreType.DMA,
        ],
    )
    def k(x_hbm, src_hbm, dst_hbm, out_hbm, agg_sh, src_v, dst_v, rows_v, sem):
        cid = lax.axis_index("c")
        sid = lax.axis_index("s")
        wid = cid * NS + sid

        zero16 = jnp.zeros((16,), jnp.float32)

        def zrow(r, carry):
            for j in range(D_AUG // 16):
                rows_v[r, pl.ds(j * 16, 16)] = zero16
            return carry

        lax.fori_loop(0, CHUNK, zrow, 0)
        row0 = sid * ROWS_PER_TILE
        for i in range(ROWS_PER_TILE // CHUNK):
            pltpu.sync_copy(rows_v, agg_sh.at[pl.ds(row0 + i * CHUNK, CHUNK)])
        rem = ROWS_PER_TILE % CHUNK
        if rem:
            pltpu.sync_copy(
                rows_v.at[pl.ds(0, rem)],
                agg_sh.at[pl.ds(row0 + (ROWS_PER_TILE // CHUNK) * CHUNK, rem)],
            )
        plsc.subcore_barrier()

        base = wid * CHUNKS_PER_W

        def outer(blk, carry):
            pltpu.sync_copy(src_hbm.at[pl.ds(base + blk * IDX_BLK, IDX_BLK)], src_v)
            pltpu.sync_copy(dst_hbm.at[pl.ds(base + blk * IDX_BLK, IDX_BLK)], dst_v)

            def step(kk, c2):
                pltpu.async_copy(x_hbm.at[src_v.at[kk]], rows_v, sem).wait()
                pltpu.sync_copy(rows_v, agg_sh.at[dst_v.at[kk]], add=True)
                return c2

            lax.fori_loop(0, IDX_BLK, step, 0)
            return carry

        lax.fori_loop(0, N_IDX_LOADS, outer, 0)
        plsc.subcore_barrier()

        pltpu.sync_copy(
            agg_sh.at[pl.ds(sid * ROWS_PER_TILE, ROWS_PER_TILE)],
            out_hbm.at[cid, pl.ds(sid * ROWS_PER_TILE, ROWS_PER_TILE)],
        )

    return k(x_aug, src2d, dst2d)


def _tc_body(agg_ref, x_ref, wlT_ref, wrT_ref, b_ref, out_ref):
    a = agg_ref[0] + agg_ref[1]
    deg = jnp.maximum(a[:, D_IN : D_IN + 1], 1.0)
    mean = a[:, :D_IN] / deg
    acc = jnp.dot(mean, wlT_ref[...], preferred_element_type=jnp.float32)
    acc += jnp.dot(x_ref[...], wrT_ref[...], preferred_element_type=jnp.float32)
    out_ref[...] = acc + b_ref[...]


def _tc_combine(agg2, x, W_l, W_r, b):
    BLK = 400
    grid = (N_NODES // BLK,)
    return pl.pallas_call(
        _tc_body,
        grid=grid,
        in_specs=[
            pl.BlockSpec((NC, BLK, D_AUG), lambda i: (0, i, 0)),
            pl.BlockSpec((BLK, D_IN), lambda i: (i, 0)),
            pl.BlockSpec((D_IN, D_OUT), lambda i: (0, 0)),
            pl.BlockSpec((D_IN, D_OUT), lambda i: (0, 0)),
            pl.BlockSpec((1, D_OUT), lambda i: (0, 0)),
        ],
        out_specs=pl.BlockSpec((BLK, D_OUT), lambda i: (i, 0)),
        out_shape=jax.ShapeDtypeStruct((N_NODES, D_OUT), jnp.float32),
    )(agg2, x, W_l.T, W_r.T, b.reshape(1, D_OUT))


def kernel(x, edge_index, W_l, W_r, b):
    src = edge_index[0].astype(jnp.int32).reshape(N_EDGES // CHUNK, CHUNK)
    dst = edge_index[1].astype(jnp.int32).reshape(N_EDGES // CHUNK, CHUNK)
    x_aug = jnp.concatenate(
        [
            x,
            jnp.ones((N_NODES, 1), jnp.float32),
            jnp.zeros((N_NODES, D_AUG - D_IN - 1), jnp.float32),
        ],
        axis=1,
    )
    agg2 = _sc_aggregate(x_aug, src, dst)
    return _tc_combine(agg2, x, W_l, W_r, b)


# speedup vs baseline: 8.9025x; 1.2106x over previous
import functools

import jax
import jax.numpy as jnp
from jax import lax
from jax.experimental import pallas as pl
from jax.experimental.pallas import tpu as pltpu
from jax.experimental.pallas import tpu_sc as plsc

N_NODES = 10000
N_EDGES = 320000
D_IN = 128
D_AUG = 144
D_OUT = 128

NC = 2
NS = 16
NW = NC * NS
CHUNK = 80
EDGES_PER_W = N_EDGES // NW
CHUNKS_PER_W = EDGES_PER_W // CHUNK
IDX_BLK = 25
N_IDX_LOADS = CHUNKS_PER_W // IDX_BLK
ROWS_PER_TILE = N_NODES // NS


def _sc_aggregate(x_aug, src2d, dst2d):
    mesh = plsc.VectorSubcoreMesh(
        core_axis_name="c", subcore_axis_name="s", num_cores=NC, num_subcores=NS
    )

    @functools.partial(
        pl.kernel,
        out_type=jax.ShapeDtypeStruct((NC, N_NODES, D_AUG), jnp.float32),
        mesh=mesh,
        compiler_params=pltpu.CompilerParams(use_tc_tiling_on_sc=False),
        scratch_types=[
            pltpu.VMEM_SHARED((N_NODES, D_AUG), jnp.float32),
            pltpu.VMEM((IDX_BLK, CHUNK), jnp.int32),
            pltpu.VMEM((IDX_BLK, CHUNK), jnp.int32),
            pltpu.VMEM((CHUNK, D_AUG), jnp.float32),
            pltpu.VMEM((CHUNK, D_AUG), jnp.float32),
            pltpu.SemaphoreType.DMA,
            pltpu.SemaphoreType.DMA,
        ],
    )
    def k(x_hbm, src_hbm, dst_hbm, out_hbm, agg_sh, src_v, dst_v, rows_a, rows_b,
          gsem, ssem):
        cid = lax.axis_index("c")
        sid = lax.axis_index("s")
        wid = cid * NS + sid

        zero16 = jnp.zeros((16,), jnp.float32)

        def zrow(r, carry):
            for j in range(D_AUG // 16):
                rows_a[r, pl.ds(j * 16, 16)] = zero16
            return carry

        lax.fori_loop(0, CHUNK, zrow, 0)
        row0 = sid * ROWS_PER_TILE
        for i in range(ROWS_PER_TILE // CHUNK):
            pltpu.sync_copy(rows_a, agg_sh.at[pl.ds(row0 + i * CHUNK, CHUNK)])
        rem = ROWS_PER_TILE % CHUNK
        if rem:
            pltpu.sync_copy(
                rows_a.at[pl.ds(0, rem)],
                agg_sh.at[pl.ds(row0 + (ROWS_PER_TILE // CHUNK) * CHUNK, rem)],
            )
        plsc.subcore_barrier()

        base = wid * CHUNKS_PER_W

        def wait_g():
            pltpu.make_async_copy(x_hbm.at[src_v.at[0]], rows_a, gsem).wait()

        def wait_s():
            pltpu.make_async_copy(rows_a, agg_sh.at[dst_v.at[0]], ssem).wait()

        def gather(kk, buf):
            pltpu.async_copy(x_hbm.at[src_v.at[kk]], buf, gsem)

        def scatter(kk, buf):
            pltpu.async_copy(buf, agg_sh.at[dst_v.at[kk]], ssem, add=True)

        def outer(blk, carry):
            lax.cond(blk > 0, wait_s, lambda: None)
            pltpu.sync_copy(src_hbm.at[pl.ds(base + blk * IDX_BLK, IDX_BLK)], src_v)
            pltpu.sync_copy(dst_hbm.at[pl.ds(base + blk * IDX_BLK, IDX_BLK)], dst_v)

            gather(0, rows_a)
            wait_g()
            scatter(0, rows_a)
            gather(1, rows_b)

            def pair(i, c2):
                k1 = 2 * i + 1
                k2 = 2 * i + 2
                wait_g()
                wait_s()
                gather(k1 + 1, rows_a)
                scatter(k1, rows_b)
                wait_g()
                wait_s()
                lax.cond(k2 < IDX_BLK - 1, lambda: gather(k2 + 1, rows_b),
                         lambda: None)
                scatter(k2, rows_a)
                return c2

            lax.fori_loop(0, (IDX_BLK - 1) // 2, pair, 0)
            return carry

        lax.fori_loop(0, N_IDX_LOADS, outer, 0)
        wait_s()
        plsc.subcore_barrier()

        pltpu.sync_copy(
            agg_sh.at[pl.ds(sid * ROWS_PER_TILE, ROWS_PER_TILE)],
            out_hbm.at[cid, pl.ds(sid * ROWS_PER_TILE, ROWS_PER_TILE)],
        )

    return k(x_aug, src2d, dst2d)


def _tc_body(agg_ref, x_ref, wlT_ref, wrT_ref, b_ref, out_ref):
    a = agg_ref[0] + agg_ref[1]
    deg = jnp.maximum(a[:, D_IN : D_IN + 1], 1.0)
    mean = a[:, :D_IN] / deg
    acc = jnp.dot(mean, wlT_ref[...], preferred_element_type=jnp.float32)
    acc += jnp.dot(x_ref[...], wrT_ref[...], preferred_element_type=jnp.float32)
    out_ref[...] = acc + b_ref[...]


def _tc_combine(agg2, x, W_l, W_r, b):
    BLK = 400
    grid = (N_NODES // BLK,)
    return pl.pallas_call(
        _tc_body,
        grid=grid,
        in_specs=[
            pl.BlockSpec((NC, BLK, D_AUG), lambda i: (0, i, 0)),
            pl.BlockSpec((BLK, D_IN), lambda i: (i, 0)),
            pl.BlockSpec((D_IN, D_OUT), lambda i: (0, 0)),
            pl.BlockSpec((D_IN, D_OUT), lambda i: (0, 0)),
            pl.BlockSpec((1, D_OUT), lambda i: (0, 0)),
        ],
        out_specs=pl.BlockSpec((BLK, D_OUT), lambda i: (i, 0)),
        out_shape=jax.ShapeDtypeStruct((N_NODES, D_OUT), jnp.float32),
    )(agg2, x, W_l.T, W_r.T, b.reshape(1, D_OUT))


def kernel(x, edge_index, W_l, W_r, b):
    src = edge_index[0].astype(jnp.int32).reshape(N_EDGES // CHUNK, CHUNK)
    dst = edge_index[1].astype(jnp.int32).reshape(N_EDGES // CHUNK, CHUNK)
    x_aug = jnp.concatenate(
        [
            x,
            jnp.ones((N_NODES, 1), jnp.float32),
            jnp.zeros((N_NODES, D_AUG - D_IN - 1), jnp.float32),
        ],
        axis=1,
    )
    agg2 = _sc_aggregate(x_aug, src, dst)
    return _tc_combine(agg2, x, W_l, W_r, b)


# speedup vs baseline: 11.2354x; 1.2621x over previous
import functools

import jax
import jax.numpy as jnp
from jax import lax
from jax.experimental import pallas as pl
from jax.experimental.pallas import tpu as pltpu
from jax.experimental.pallas import tpu_sc as plsc

N_NODES = 10000
N_EDGES = 320000
D_IN = 128
D_AUG = 144
D_OUT = 128

NC = 2
NS = 16
NW = NC * NS
CHUNK = 80
EDGES_PER_W = N_EDGES // NW
CHUNKS_PER_W = EDGES_PER_W // CHUNK
IDX_BLK = 25
N_IDX_LOADS = CHUNKS_PER_W // IDX_BLK
ROWS_PER_TILE = N_NODES // NS


def _sc_aggregate(x_aug, src2d, dst2d):
    mesh = plsc.VectorSubcoreMesh(
        core_axis_name="c", subcore_axis_name="s", num_cores=NC, num_subcores=NS
    )

    @functools.partial(
        pl.kernel,
        out_type=jax.ShapeDtypeStruct((NC, N_NODES, D_AUG), jnp.float32),
        mesh=mesh,
        compiler_params=pltpu.CompilerParams(use_tc_tiling_on_sc=False),
        scratch_types=[
            pltpu.VMEM_SHARED((N_NODES, D_AUG), jnp.float32),
            pltpu.VMEM((IDX_BLK, CHUNK), jnp.int32),
            pltpu.VMEM((IDX_BLK, CHUNK), jnp.int32),
            pltpu.VMEM((CHUNK, D_AUG), jnp.float32),
            pltpu.VMEM((CHUNK, D_AUG), jnp.float32),
            pltpu.VMEM((CHUNK, D_AUG), jnp.float32),
            pltpu.SemaphoreType.DMA,
            pltpu.SemaphoreType.DMA,
            pltpu.SemaphoreType.DMA,
            pltpu.SemaphoreType.DMA,
        ],
    )
    def k(x_hbm, src_hbm, dst_hbm, out_hbm, agg_sh, src_v, dst_v, rows_a, rows_b,
          rows_c, gsa, gsb, gsc, ssem):
        cid = lax.axis_index("c")
        sid = lax.axis_index("s")
        wid = cid * NS + sid

        zero16 = jnp.zeros((16,), jnp.float32)

        def zrow(r, carry):
            for j in range(D_AUG // 16):
                rows_a[r, pl.ds(j * 16, 16)] = zero16
            return carry

        lax.fori_loop(0, CHUNK, zrow, 0)
        row0 = sid * ROWS_PER_TILE
        for i in range(ROWS_PER_TILE // CHUNK):
            pltpu.sync_copy(rows_a, agg_sh.at[pl.ds(row0 + i * CHUNK, CHUNK)])
        rem = ROWS_PER_TILE % CHUNK
        if rem:
            pltpu.sync_copy(
                rows_a.at[pl.ds(0, rem)],
                agg_sh.at[pl.ds(row0 + (ROWS_PER_TILE // CHUNK) * CHUNK, rem)],
            )
        plsc.subcore_barrier()

        base = wid * CHUNKS_PER_W

        def wait_g(sem):
            pltpu.make_async_copy(x_hbm.at[src_v.at[0]], rows_a, sem).wait()

        def wait_s():
            pltpu.make_async_copy(rows_a, agg_sh.at[dst_v.at[0]], ssem).wait()

        def gather(kk, buf, sem):
            pltpu.async_copy(x_hbm.at[src_v.at[kk]], buf, sem)

        def scatter(kk, buf):
            pltpu.async_copy(buf, agg_sh.at[dst_v.at[kk]], ssem, add=True)

        def outer(blk, carry):
            lax.cond(blk > 0, wait_s, lambda: None)
            pltpu.sync_copy(src_hbm.at[pl.ds(base + blk * IDX_BLK, IDX_BLK)], src_v)
            pltpu.sync_copy(dst_hbm.at[pl.ds(base + blk * IDX_BLK, IDX_BLK)], dst_v)

            gather(0, rows_a, gsa)
            gather(1, rows_b, gsb)
            wait_g(gsa)
            gather(2, rows_c, gsc)
            scatter(0, rows_a)

            def triple(i, c2):
                for d, (nxt, nsem, cur, csem) in (
                    (1, (rows_a, gsa, rows_b, gsb)),
                    (2, (rows_b, gsb, rows_c, gsc)),
                    (3, (rows_c, gsc, rows_a, gsa)),
                ):
                    kk = 3 * i + d
                    wait_g(csem)
                    wait_s()
                    lax.cond(kk + 2 <= IDX_BLK - 1,
                             (lambda kk=kk, nxt=nxt, nsem=nsem:
                              gather(kk + 2, nxt, nsem)),
                             lambda: None)
                    scatter(kk, cur)
                return c2

            lax.fori_loop(0, (IDX_BLK - 1) // 3, triple, 0)
            return carry

        lax.fori_loop(0, N_IDX_LOADS, outer, 0)
        wait_s()
        plsc.subcore_barrier()

        pltpu.sync_copy(
            agg_sh.at[pl.ds(sid * ROWS_PER_TILE, ROWS_PER_TILE)],
            out_hbm.at[cid, pl.ds(sid * ROWS_PER_TILE, ROWS_PER_TILE)],
        )

    return k(x_aug, src2d, dst2d)


def _tc_body(agg_ref, x_ref, wlT_ref, wrT_ref, b_ref, out_ref):
    a = agg_ref[0] + agg_ref[1]
    deg = jnp.maximum(a[:, D_IN : D_IN + 1], 1.0)
    mean = a[:, :D_IN] / deg
    acc = jnp.dot(mean, wlT_ref[...], preferred_element_type=jnp.float32)
    acc += jnp.dot(x_ref[...], wrT_ref[...], preferred_element_type=jnp.float32)
    out_ref[...] = acc + b_ref[...]


def _tc_combine(agg2, x, W_l, W_r, b):
    BLK = 400
    grid = (N_NODES // BLK,)
    return pl.pallas_call(
        _tc_body,
        grid=grid,
        in_specs=[
            pl.BlockSpec((NC, BLK, D_AUG), lambda i: (0, i, 0)),
            pl.BlockSpec((BLK, D_IN), lambda i: (i, 0)),
            pl.BlockSpec((D_IN, D_OUT), lambda i: (0, 0)),
            pl.BlockSpec((D_IN, D_OUT), lambda i: (0, 0)),
            pl.BlockSpec((1, D_OUT), lambda i: (0, 0)),
        ],
        out_specs=pl.BlockSpec((BLK, D_OUT), lambda i: (i, 0)),
        out_shape=jax.ShapeDtypeStruct((N_NODES, D_OUT), jnp.float32),
    )(agg2, x, W_l.T, W_r.T, b.reshape(1, D_OUT))


def kernel(x, edge_index, W_l, W_r, b):
    src = edge_index[0].astype(jnp.int32).reshape(N_EDGES // CHUNK, CHUNK)
    dst = edge_index[1].astype(jnp.int32).reshape(N_EDGES // CHUNK, CHUNK)
    x_aug = jnp.concatenate(
        [
            x,
            jnp.ones((N_NODES, 1), jnp.float32),
            jnp.zeros((N_NODES, D_AUG - D_IN - 1), jnp.float32),
        ],
        axis=1,
    )
    agg2 = _sc_aggregate(x_aug, src, dst)
    return _tc_combine(agg2, x, W_l, W_r, b)


# speedup vs baseline: 13.5781x; 1.2085x over previous
import functools

import jax
import jax.numpy as jnp
from jax import lax
from jax.experimental import pallas as pl
from jax.experimental.pallas import tpu as pltpu
from jax.experimental.pallas import tpu_sc as plsc

N_NODES = 10000
N_EDGES = 320000
D_IN = 128
D_OUT = 128
DEG_W = 16

NC = 2
NS = 16
NW = NC * NS
CHUNK = 80
EDGES_PER_W = N_EDGES // NW
CHUNKS_PER_W = EDGES_PER_W // CHUNK
IDX_BLK = 25
N_IDX_LOADS = CHUNKS_PER_W // IDX_BLK
ROWS_PER_TILE = N_NODES // NS


def _sc_aggregate(x, src, dst):
    mesh = plsc.VectorSubcoreMesh(
        core_axis_name="c", subcore_axis_name="s", num_cores=NC, num_subcores=NS
    )

    @functools.partial(
        pl.kernel,
        out_type=(
            jax.ShapeDtypeStruct((NC, N_NODES, D_IN), jnp.float32),
            jax.ShapeDtypeStruct((NC, N_NODES, DEG_W), jnp.float32),
        ),
        mesh=mesh,
        compiler_params=pltpu.CompilerParams(use_tc_tiling_on_sc=False),
        scratch_types=[
            pltpu.VMEM_SHARED((N_NODES, D_IN), jnp.float32),
            pltpu.VMEM_SHARED((N_NODES, DEG_W), jnp.float32),
            pltpu.VMEM((IDX_BLK * CHUNK,), jnp.int32),
            pltpu.VMEM((IDX_BLK * CHUNK,), jnp.int32),
            pltpu.VMEM((CHUNK, D_IN), jnp.float32),
            pltpu.VMEM((CHUNK, D_IN), jnp.float32),
            pltpu.VMEM((CHUNK, D_IN), jnp.float32),
            pltpu.VMEM((CHUNK, DEG_W), jnp.float32),
            pltpu.VMEM((CHUNK, DEG_W), jnp.float32),
            pltpu.SemaphoreType.DMA,
            pltpu.SemaphoreType.DMA,
            pltpu.SemaphoreType.DMA,
            pltpu.SemaphoreType.DMA,
            pltpu.SemaphoreType.DMA,
        ],
    )
    def k(x_hbm, src_hbm, dst_hbm, out_hbm, deg_hbm, agg_sh, deg_sh, src_v,
          dst_v, rows_a, rows_b, rows_c, ones_v, zdeg_v, gsa, gsb, gsc, ssem,
          dsem):
        cid = lax.axis_index("c")
        sid = lax.axis_index("s")
        wid = cid * NS + sid

        zero16 = jnp.zeros((16,), jnp.float32)
        one16 = jnp.ones((16,), jnp.float32)

        def fill(r, carry):
            for j in range(D_IN // 16):
                rows_a[r, pl.ds(j * 16, 16)] = zero16
            ones_v[r, pl.ds(0, DEG_W)] = one16
            zdeg_v[r, pl.ds(0, DEG_W)] = zero16
            return carry

        lax.fori_loop(0, CHUNK, fill, 0)
        row0 = sid * ROWS_PER_TILE
        nfull = ROWS_PER_TILE // CHUNK
        for i in range(nfull):
            pltpu.sync_copy(rows_a, agg_sh.at[pl.ds(row0 + i * CHUNK, CHUNK)])
            pltpu.sync_copy(zdeg_v, deg_sh.at[pl.ds(row0 + i * CHUNK, CHUNK)])
        rem = ROWS_PER_TILE % CHUNK
        if rem:
            pltpu.sync_copy(rows_a.at[pl.ds(0, rem)],
                            agg_sh.at[pl.ds(row0 + nfull * CHUNK, rem)])
            pltpu.sync_copy(zdeg_v.at[pl.ds(0, rem)],
                            deg_sh.at[pl.ds(row0 + nfull * CHUNK, rem)])
        plsc.subcore_barrier()

        base = wid * CHUNKS_PER_W

        def wait_g(sem):
            pltpu.make_async_copy(x_hbm.at[src_v.at[pl.ds(0, CHUNK)]], rows_a, sem).wait()

        def wait_s():
            pltpu.make_async_copy(rows_a, agg_sh.at[dst_v.at[pl.ds(0, CHUNK)]], ssem).wait()

        def wait_d():
            pltpu.make_async_copy(ones_v, deg_sh.at[dst_v.at[pl.ds(0, CHUNK)]], dsem).wait()

        def drain_deg():
            def w(_, c):
                wait_d()
                return c
            lax.fori_loop(0, IDX_BLK, w, 0)

        def gather(kk, buf, sem):
            pltpu.async_copy(x_hbm.at[src_v.at[pl.ds(kk * CHUNK, CHUNK)]], buf, sem)

        def scatter(kk, buf):
            pltpu.async_copy(buf, agg_sh.at[dst_v.at[pl.ds(kk * CHUNK, CHUNK)]], ssem, add=True)
            pltpu.async_copy(ones_v, deg_sh.at[dst_v.at[pl.ds(kk * CHUNK, CHUNK)]], dsem, add=True)

        def outer(blk, carry):
            lax.cond(blk > 0, lambda: (wait_s(), drain_deg())[1], lambda: None)
            e0 = (base + blk * IDX_BLK) * CHUNK
            pltpu.sync_copy(src_hbm.at[pl.ds(e0, IDX_BLK * CHUNK)], src_v)
            pltpu.sync_copy(dst_hbm.at[pl.ds(e0, IDX_BLK * CHUNK)], dst_v)

            gather(0, rows_a, gsa)
            gather(1, rows_b, gsb)
            wait_g(gsa)
            gather(2, rows_c, gsc)
            scatter(0, rows_a)

            def triple(i, c2):
                for d, (nxt, nsem, cur, csem) in (
                    (1, (rows_a, gsa, rows_b, gsb)),
                    (2, (rows_b, gsb, rows_c, gsc)),
                    (3, (rows_c, gsc, rows_a, gsa)),
                ):
                    kk = 3 * i + d
                    wait_g(csem)
                    wait_s()
                    lax.cond(kk + 2 <= IDX_BLK - 1,
                             (lambda kk=kk, nxt=nxt, nsem=nsem:
                              gather(kk + 2, nxt, nsem)),
                             lambda: None)
                    scatter(kk, cur)
                return c2

            lax.fori_loop(0, (IDX_BLK - 1) // 3, triple, 0)
            return carry

        lax.fori_loop(0, N_IDX_LOADS, outer, 0)
        wait_s()
        drain_deg()
        plsc.subcore_barrier()

        pltpu.sync_copy(
            agg_sh.at[pl.ds(row0, ROWS_PER_TILE)],
            out_hbm.at[cid, pl.ds(row0, ROWS_PER_TILE)],
        )
        pltpu.sync_copy(
            deg_sh.at[pl.ds(row0, ROWS_PER_TILE)],
            deg_hbm.at[cid, pl.ds(row0, ROWS_PER_TILE)],
        )

    return k(x, src, dst)


def _tc_body(agg_ref, deg_ref, x_ref, wlT_ref, wrT_ref, b_ref, out_ref):
    a = agg_ref[0] + agg_ref[1]
    deg = jnp.maximum(deg_ref[0, :, 0:1] + deg_ref[1, :, 0:1], 1.0)
    mean = a / deg
    acc = jnp.dot(mean, wlT_ref[...], preferred_element_type=jnp.float32)
    acc += jnp.dot(x_ref[...], wrT_ref[...], preferred_element_type=jnp.float32)
    out_ref[...] = acc + b_ref[...]


def _tc_combine(agg2, deg2, x, W_l, W_r, b):
    BLK = 400
    grid = (N_NODES // BLK,)
    return pl.pallas_call(
        _tc_body,
        grid=grid,
        in_specs=[
            pl.BlockSpec((NC, BLK, D_IN), lambda i: (0, i, 0)),
            pl.BlockSpec((NC, BLK, DEG_W), lambda i: (0, i, 0)),
            pl.BlockSpec((BLK, D_IN), lambda i: (i, 0)),
            pl.BlockSpec((D_IN, D_OUT), lambda i: (0, 0)),
            pl.BlockSpec((D_IN, D_OUT), lambda i: (0, 0)),
            pl.BlockSpec((1, D_OUT), lambda i: (0, 0)),
        ],
        out_specs=pl.BlockSpec((BLK, D_OUT), lambda i: (i, 0)),
        out_shape=jax.ShapeDtypeStruct((N_NODES, D_OUT), jnp.float32),
    )(agg2, deg2, x, W_l.T, W_r.T, b.reshape(1, D_OUT))


def kernel(x, edge_index, W_l, W_r, b):
    src = edge_index[0].astype(jnp.int32)
    dst = edge_index[1].astype(jnp.int32)
    agg2, deg2 = _sc_aggregate(x, src, dst)
    return _tc_combine(agg2, deg2, x, W_l, W_r, b)


# speedup vs baseline: 15.2231x; 1.1212x over previous
import functools

import jax
import jax.numpy as jnp
from jax import lax
from jax.experimental import pallas as pl
from jax.experimental.pallas import tpu as pltpu
from jax.experimental.pallas import tpu_sc as plsc

N_NODES = 10000
N_EDGES = 320000
D_IN = 128
D_OUT = 128
DEG_W = 16

NC = 2
NS = 16
NW = NC * NS
CHUNK = 80
EDGES_PER_W = N_EDGES // NW
CHUNKS_PER_W = EDGES_PER_W // CHUNK
IDX_BLK = 25
N_IDX_LOADS = CHUNKS_PER_W // IDX_BLK
ROWS_PER_TILE = N_NODES // NS


def _sc_aggregate(x, edge_index):
    mesh = plsc.VectorSubcoreMesh(
        core_axis_name="c", subcore_axis_name="s", num_cores=NC, num_subcores=NS
    )

    @functools.partial(
        pl.kernel,
        out_type=(
            jax.ShapeDtypeStruct((NC, N_NODES, D_IN), jnp.float32),
            jax.ShapeDtypeStruct((NC, N_NODES, DEG_W), jnp.float32),
        ),
        mesh=mesh,
        compiler_params=pltpu.CompilerParams(use_tc_tiling_on_sc=False),
        scratch_types=[
            pltpu.VMEM_SHARED((N_NODES, D_IN), jnp.float32),
            pltpu.VMEM_SHARED((N_NODES, DEG_W), jnp.float32),
            pltpu.VMEM((IDX_BLK * CHUNK,), jnp.int32),
            pltpu.VMEM((IDX_BLK * CHUNK,), jnp.int32),
            pltpu.VMEM((CHUNK, D_IN), jnp.float32),
            pltpu.VMEM((CHUNK, D_IN), jnp.float32),
            pltpu.VMEM((CHUNK, D_IN), jnp.float32),
            pltpu.VMEM((CHUNK, DEG_W), jnp.float32),
            pltpu.VMEM((CHUNK, DEG_W), jnp.float32),
            pltpu.SemaphoreType.DMA,
            pltpu.SemaphoreType.DMA,
            pltpu.SemaphoreType.DMA,
            pltpu.SemaphoreType.DMA,
            pltpu.SemaphoreType.DMA,
            pltpu.SemaphoreType.DMA,
        ],
    )
    def k(x_hbm, ei_hbm, out_hbm, deg_hbm, agg_sh, deg_sh, src_v,
          dst_v, rows_a, rows_b, rows_c, ones_v, zdeg_v, gsa, gsb, gsc, ssa,
          ssb, dsem):
        cid = lax.axis_index("c")
        sid = lax.axis_index("s")
        wid = cid * NS + sid

        zero16 = jnp.zeros((16,), jnp.float32)
        one16 = jnp.ones((16,), jnp.float32)

        def fill(r, carry):
            for j in range(D_IN // 16):
                rows_a[r, pl.ds(j * 16, 16)] = zero16
            ones_v[r, pl.ds(0, DEG_W)] = one16
            zdeg_v[r, pl.ds(0, DEG_W)] = zero16
            return carry

        lax.fori_loop(0, CHUNK, fill, 0)
        row0 = sid * ROWS_PER_TILE
        nfull = ROWS_PER_TILE // CHUNK
        for i in range(nfull):
            pltpu.sync_copy(rows_a, agg_sh.at[pl.ds(row0 + i * CHUNK, CHUNK)])
            pltpu.sync_copy(zdeg_v, deg_sh.at[pl.ds(row0 + i * CHUNK, CHUNK)])
        rem = ROWS_PER_TILE % CHUNK
        if rem:
            pltpu.sync_copy(rows_a.at[pl.ds(0, rem)],
                            agg_sh.at[pl.ds(row0 + nfull * CHUNK, rem)])
            pltpu.sync_copy(zdeg_v.at[pl.ds(0, rem)],
                            deg_sh.at[pl.ds(row0 + nfull * CHUNK, rem)])
        plsc.subcore_barrier()

        base = wid * CHUNKS_PER_W

        def wait_g(sem):
            pltpu.make_async_copy(x_hbm.at[src_v.at[pl.ds(0, CHUNK)]], rows_a, sem).wait()

        def wait_s(sem):
            pltpu.make_async_copy(rows_a, agg_sh.at[dst_v.at[pl.ds(0, CHUNK)]], sem).wait()

        def wait_d():
            pltpu.make_async_copy(ones_v, deg_sh.at[dst_v.at[pl.ds(0, CHUNK)]], dsem).wait()

        def drain_deg():
            def w(_, c):
                wait_d()
                return c
            lax.fori_loop(0, IDX_BLK, w, 0)

        def gather(kk, buf, sem):
            pltpu.async_copy(x_hbm.at[src_v.at[pl.ds(kk * CHUNK, CHUNK)]], buf, sem)

        def scatter(kk, buf, sem):
            pltpu.async_copy(buf, agg_sh.at[dst_v.at[pl.ds(kk * CHUNK, CHUNK)]], sem, add=True)
            pltpu.async_copy(ones_v, deg_sh.at[dst_v.at[pl.ds(kk * CHUNK, CHUNK)]], dsem, add=True)

        def outer(blk, carry):
            lax.cond(blk > 0, lambda: (wait_s(ssa), drain_deg())[1], lambda: None)
            e0 = (base + blk * IDX_BLK) * CHUNK
            pltpu.sync_copy(ei_hbm.at[0, pl.ds(e0, IDX_BLK * CHUNK)], src_v)
            pltpu.sync_copy(ei_hbm.at[1, pl.ds(e0, IDX_BLK * CHUNK)], dst_v)

            gather(0, rows_a, gsa)
            gather(1, rows_b, gsb)
            wait_g(gsa)
            gather(2, rows_c, gsc)
            scatter(0, rows_a, ssa)

            def sextet(i, c2):
                for d, (nxt, nsem, cur, csem) in (
                    (1, (rows_a, gsa, rows_b, gsb)),
                    (2, (rows_b, gsb, rows_c, gsc)),
                    (3, (rows_c, gsc, rows_a, gsa)),
                    (4, (rows_a, gsa, rows_b, gsb)),
                    (5, (rows_b, gsb, rows_c, gsc)),
                    (6, (rows_c, gsc, rows_a, gsa)),
                ):
                    kk = 6 * i + d
                    mine = ssb if d % 2 else ssa
                    prev = ssa if d % 2 else ssb
                    wait_g(csem)
                    scatter(kk, cur, mine)
                    wait_s(prev)
                    lax.cond(kk + 2 <= IDX_BLK - 1,
                             (lambda kk=kk, nxt=nxt, nsem=nsem:
                              gather(kk + 2, nxt, nsem)),
                             lambda: None)
                return c2

            lax.fori_loop(0, (IDX_BLK - 1) // 6, sextet, 0)
            return carry

        lax.fori_loop(0, N_IDX_LOADS, outer, 0)
        wait_s(ssa)
        drain_deg()
        plsc.subcore_barrier()

        pltpu.sync_copy(
            agg_sh.at[pl.ds(row0, ROWS_PER_TILE)],
            out_hbm.at[cid, pl.ds(row0, ROWS_PER_TILE)],
        )
        pltpu.sync_copy(
            deg_sh.at[pl.ds(row0, ROWS_PER_TILE)],
            deg_hbm.at[cid, pl.ds(row0, ROWS_PER_TILE)],
        )

    return k(x, edge_index)


def _tc_body(agg_ref, deg_ref, x_ref, wlT_ref, wrT_ref, b_ref, out_ref):
    a = agg_ref[0] + agg_ref[1]
    deg = jnp.maximum(deg_ref[0, :, 0:1] + deg_ref[1, :, 0:1], 1.0)
    mean = a / deg
    acc = jnp.dot(mean, wlT_ref[...], preferred_element_type=jnp.float32)
    acc += jnp.dot(x_ref[...], wrT_ref[...], preferred_element_type=jnp.float32)
    out_ref[...] = acc + b_ref[...]


def _tc_combine(agg2, deg2, x, W_l, W_r, b):
    BLK = 2000
    grid = (N_NODES // BLK,)
    return pl.pallas_call(
        _tc_body,
        grid=grid,
        in_specs=[
            pl.BlockSpec((NC, BLK, D_IN), lambda i: (0, i, 0)),
            pl.BlockSpec((NC, BLK, DEG_W), lambda i: (0, i, 0)),
            pl.BlockSpec((BLK, D_IN), lambda i: (i, 0)),
            pl.BlockSpec((D_IN, D_OUT), lambda i: (0, 0)),
            pl.BlockSpec((D_IN, D_OUT), lambda i: (0, 0)),
            pl.BlockSpec((1, D_OUT), lambda i: (0, 0)),
        ],
        out_specs=pl.BlockSpec((BLK, D_OUT), lambda i: (i, 0)),
        out_shape=jax.ShapeDtypeStruct((N_NODES, D_OUT), jnp.float32),
    )(agg2, deg2, x, W_l.T, W_r.T, b.reshape(1, D_OUT))


def kernel(x, edge_index, W_l, W_r, b):
    agg2, deg2 = _sc_aggregate(x, edge_index.astype(jnp.int32))
    return _tc_combine(agg2, deg2, x, W_l, W_r, b)


# speedup vs baseline: 15.5251x; 1.0198x over previous
import functools

import jax
import jax.numpy as jnp
from jax import lax
from jax.experimental import pallas as pl
from jax.experimental.pallas import tpu as pltpu
from jax.experimental.pallas import tpu_sc as plsc

N_NODES = 10000
N_EDGES = 320000
D_IN = 128
D_OUT = 128
DEG_W = 16

NC = 2
NS = 16
NW = NC * NS
CHUNK = 80
EDGES_PER_W = N_EDGES // NW
CHUNKS_PER_W = EDGES_PER_W // CHUNK
IDX_BLK = 25
N_IDX_LOADS = CHUNKS_PER_W // IDX_BLK
ROWS_PER_TILE = N_NODES // NS


def _sc_aggregate(x, edge_index):
    mesh = plsc.VectorSubcoreMesh(
        core_axis_name="c", subcore_axis_name="s", num_cores=NC, num_subcores=NS
    )

    @functools.partial(
        pl.kernel,
        out_type=(
            jax.ShapeDtypeStruct((NC, N_NODES, D_IN), jnp.float32),
            jax.ShapeDtypeStruct((NC, N_NODES, DEG_W), jnp.float32),
        ),
        mesh=mesh,
        compiler_params=pltpu.CompilerParams(use_tc_tiling_on_sc=False),
        scratch_types=[
            pltpu.VMEM_SHARED((N_NODES, D_IN), jnp.float32),
            pltpu.VMEM_SHARED((N_NODES, DEG_W), jnp.float32),
            pltpu.VMEM((IDX_BLK * CHUNK,), jnp.int32),
            pltpu.VMEM((IDX_BLK * CHUNK,), jnp.int32),
            pltpu.VMEM((CHUNK, D_IN), jnp.float32),
            pltpu.VMEM((CHUNK, D_IN), jnp.float32),
            pltpu.VMEM((CHUNK, D_IN), jnp.float32),
            pltpu.VMEM((CHUNK, DEG_W), jnp.float32),
            pltpu.VMEM((CHUNK, DEG_W), jnp.float32),
            pltpu.SemaphoreType.DMA,
            pltpu.SemaphoreType.DMA,
            pltpu.SemaphoreType.DMA,
            pltpu.SemaphoreType.DMA,
            pltpu.SemaphoreType.DMA,
        ],
    )
    def k(x_hbm, ei_hbm, out_hbm, deg_hbm, agg_sh, deg_sh, src_v,
          dst_v, rows_a, rows_b, rows_c, ones_v, zdeg_v, gsa, gsb, gsc, ssa,
          dsem):
        cid = lax.axis_index("c")
        sid = lax.axis_index("s")
        wid = cid * NS + sid

        zero16 = jnp.zeros((16,), jnp.float32)
        one16 = jnp.ones((16,), jnp.float32)

        def fill(r, carry):
            for j in range(D_IN // 16):
                rows_a[r, pl.ds(j * 16, 16)] = zero16
            ones_v[r, pl.ds(0, DEG_W)] = one16
            zdeg_v[r, pl.ds(0, DEG_W)] = zero16
            return carry

        lax.fori_loop(0, CHUNK, fill, 0)
        row0 = sid * ROWS_PER_TILE
        nfull = ROWS_PER_TILE // CHUNK
        for i in range(nfull):
            pltpu.sync_copy(rows_a, agg_sh.at[pl.ds(row0 + i * CHUNK, CHUNK)])
            pltpu.sync_copy(zdeg_v, deg_sh.at[pl.ds(row0 + i * CHUNK, CHUNK)])
        rem = ROWS_PER_TILE % CHUNK
        if rem:
            pltpu.sync_copy(rows_a.at[pl.ds(0, rem)],
                            agg_sh.at[pl.ds(row0 + nfull * CHUNK, rem)])
            pltpu.sync_copy(zdeg_v.at[pl.ds(0, rem)],
                            deg_sh.at[pl.ds(row0 + nfull * CHUNK, rem)])
        plsc.subcore_barrier()

        base = wid * CHUNKS_PER_W

        def wait_g(sem):
            pltpu.make_async_copy(x_hbm.at[src_v.at[pl.ds(0, CHUNK)]], rows_a, sem).wait()

        def wait_s(sem):
            pltpu.make_async_copy(rows_a, agg_sh.at[dst_v.at[pl.ds(0, CHUNK)]], sem).wait()

        def wait_d():
            pltpu.make_async_copy(ones_v, deg_sh.at[dst_v.at[pl.ds(0, CHUNK)]], dsem).wait()

        def drain_deg():
            def w(_, c):
                wait_d()
                return c
            lax.fori_loop(0, IDX_BLK, w, 0)

        def gather(kk, buf, sem):
            pltpu.async_copy(x_hbm.at[src_v.at[pl.ds(kk * CHUNK, CHUNK)]], buf, sem)

        def scatter(kk, buf, sem):
            pltpu.async_copy(buf, agg_sh.at[dst_v.at[pl.ds(kk * CHUNK, CHUNK)]], sem, add=True)
            pltpu.async_copy(ones_v, deg_sh.at[dst_v.at[pl.ds(kk * CHUNK, CHUNK)]], dsem, add=True)

        def outer(blk, carry):
            lax.cond(blk > 0, lambda: (wait_s(ssa), drain_deg())[1], lambda: None)
            e0 = (base + blk * IDX_BLK) * CHUNK
            pltpu.sync_copy(ei_hbm.at[0, pl.ds(e0, IDX_BLK * CHUNK)], src_v)
            pltpu.sync_copy(ei_hbm.at[1, pl.ds(e0, IDX_BLK * CHUNK)], dst_v)

            gather(0, rows_a, gsa)
            gather(1, rows_b, gsb)
            wait_g(gsa)
            gather(2, rows_c, gsc)
            scatter(0, rows_a, ssa)

            def triple(i, c2):
                for d, (nxt, nsem, cur, csem) in (
                    (1, (rows_a, gsa, rows_b, gsb)),
                    (2, (rows_b, gsb, rows_c, gsc)),
                    (3, (rows_c, gsc, rows_a, gsa)),
                ):
                    kk = 3 * i + d
                    wait_g(csem)
                    wait_s(ssa)
                    lax.cond(kk + 2 <= IDX_BLK - 1,
                             (lambda kk=kk, nxt=nxt, nsem=nsem:
                              gather(kk + 2, nxt, nsem)),
                             lambda: None)
                    scatter(kk, cur, ssa)
                return c2

            lax.fori_loop(0, (IDX_BLK - 1) // 3, triple, 0)
            return carry

        lax.fori_loop(0, N_IDX_LOADS, outer, 0)
        wait_s(ssa)
        drain_deg()
        plsc.subcore_barrier()

        pltpu.sync_copy(
            agg_sh.at[pl.ds(row0, ROWS_PER_TILE)],
            out_hbm.at[cid, pl.ds(row0, ROWS_PER_TILE)],
        )
        pltpu.sync_copy(
            deg_sh.at[pl.ds(row0, ROWS_PER_TILE)],
            deg_hbm.at[cid, pl.ds(row0, ROWS_PER_TILE)],
        )

    return k(x, edge_index)


def _tc_body(agg_ref, deg_ref, x_ref, wlT_ref, wrT_ref, b_ref, out_ref):
    a = agg_ref[0] + agg_ref[1]
    deg = jnp.maximum(deg_ref[0, :, 0:1] + deg_ref[1, :, 0:1], 1.0)
    mean = a / deg
    acc = jnp.dot(mean, wlT_ref[...], preferred_element_type=jnp.float32)
    acc += jnp.dot(x_ref[...], wrT_ref[...], preferred_element_type=jnp.float32)
    out_ref[...] = acc + b_ref[...]


def _tc_combine(agg2, deg2, x, W_l, W_r, b):
    BLK = 2000
    grid = (N_NODES // BLK,)
    return pl.pallas_call(
        _tc_body,
        grid=grid,
        in_specs=[
            pl.BlockSpec((NC, BLK, D_IN), lambda i: (0, i, 0)),
            pl.BlockSpec((NC, BLK, DEG_W), lambda i: (0, i, 0)),
            pl.BlockSpec((BLK, D_IN), lambda i: (i, 0)),
            pl.BlockSpec((D_IN, D_OUT), lambda i: (0, 0)),
            pl.BlockSpec((D_IN, D_OUT), lambda i: (0, 0)),
            pl.BlockSpec((1, D_OUT), lambda i: (0, 0)),
        ],
        out_specs=pl.BlockSpec((BLK, D_OUT), lambda i: (i, 0)),
        out_shape=jax.ShapeDtypeStruct((N_NODES, D_OUT), jnp.float32),
    )(agg2, deg2, x, W_l.T, W_r.T, b.reshape(1, D_OUT))


def kernel(x, edge_index, W_l, W_r, b):
    agg2, deg2 = _sc_aggregate(x, edge_index.astype(jnp.int32))
    return _tc_combine(agg2, deg2, x, W_l, W_r, b)
